# baseline (speedup 1.0000x reference)
import jax
import jax.numpy as jnp
from jax.experimental import pallas as pl

_D = 4096
_SEQ = 8192
_BATCH = 4
_NUM_LAYERS = 32
_SBLK = 256


def _pool_kernel(x_ref, o_ref):
    i = pl.program_id(0)

    @pl.when(i == 0)
    def _():
        o_ref[...] = jnp.zeros_like(o_ref)

    o_ref[...] += jnp.sum(x_ref[...], axis=1)


def _router_kernel(p_ref, w1_ref, b1_ref, g1_ref, be1_ref,
                   w2_ref, b2_ref, g2_ref, be2_ref,
                   w3_ref, b3_ref, o_ref):
    pooled = p_ref[...] * (1.0 / _SEQ)

    def _ln(h, g, b, eps=1e-5):
        m = jnp.mean(h, axis=-1, keepdims=True)
        v = jnp.mean((h - m) ** 2, axis=-1, keepdims=True)
        return (h - m) / jnp.sqrt(v + eps) * g + b

    h = jax.lax.dot_general(pooled, w1_ref[...], (((1,), (1,)), ((), ())),
                            preferred_element_type=jnp.float32) + b1_ref[...]
    h = jax.nn.relu(_ln(h, g1_ref[...], be1_ref[...]))
    h = jax.lax.dot_general(h, w2_ref[...], (((1,), (1,)), ((), ())),
                            preferred_element_type=jnp.float32) + b2_ref[...]
    h = jax.nn.relu(_ln(h, g2_ref[...], be2_ref[...]))
    scores = jax.lax.dot_general(h, w3_ref[...], (((1,), (1,)), ((), ())),
                                 preferred_element_type=jnp.float32) + b3_ref[...]

    scaled = scores - jnp.max(scores, axis=-1, keepdims=True)
    e = jnp.exp(scaled - jnp.max(scaled, axis=-1, keepdims=True))
    probs = e / jnp.sum(e, axis=-1, keepdims=True)

    pa = probs[:, :, None]
    pb = probs[:, None, :]
    idx = jax.lax.broadcasted_iota(jnp.int32, (1, _NUM_LAYERS, _NUM_LAYERS), 1)
    jdx = jax.lax.broadcasted_iota(jnp.int32, (1, _NUM_LAYERS, _NUM_LAYERS), 2)
    beats = (pb > pa) | ((pb == pa) & (jdx < idx))
    nbeat = jnp.sum(beats.astype(jnp.int32), axis=-1)
    o_ref[...] = (nbeat < 8).astype(jnp.float32)


def kernel(x, W1, b1, g1, be1, W2, b2, g2, be2, W3, b3):
    pooled_sum = pl.pallas_call(
        _pool_kernel,
        grid=(_SEQ // _SBLK,),
        in_specs=[pl.BlockSpec((_BATCH, _SBLK, _D), lambda i: (0, i, 0))],
        out_specs=pl.BlockSpec((_BATCH, _D), lambda i: (0, 0)),
        out_shape=jax.ShapeDtypeStruct((_BATCH, _D), jnp.float32),
    )(x)

    full = lambda s: pl.BlockSpec(s, lambda: tuple(0 for _ in s))
    mask = pl.pallas_call(
        _router_kernel,
        in_specs=[full(pooled_sum.shape), full(W1.shape), full(b1.shape),
                  full(g1.shape), full(be1.shape), full(W2.shape),
                  full(b2.shape), full(g2.shape), full(be2.shape),
                  full(W3.shape), full(b3.shape)],
        out_specs=full((_BATCH, _NUM_LAYERS)),
        out_shape=jax.ShapeDtypeStruct((_BATCH, _NUM_LAYERS), jnp.float32),
    )(pooled_sum, W1, b1, g1, be1, W2, b2, g2, be2, W3, b3)
    return mask


# speedup vs baseline: 1.0870x; 1.0870x over previous
import jax
import jax.numpy as jnp
from jax.experimental import pallas as pl
from jax.experimental.pallas import tpu as pltpu

_D = 4096
_SEQ = 8192
_BATCH = 4
_NUM_LAYERS = 32
_SBLK = 128
_NSTEPS = _SEQ // _SBLK


def _fused_kernel(x_ref, w1_ref, b1_ref, g1_ref, be1_ref,
                  w2_ref, b2_ref, g2_ref, be2_ref,
                  w3_ref, b3_ref, o_ref, acc_ref):
    i = pl.program_id(0)
    part = jnp.sum(x_ref[...], axis=1)

    @pl.when(i == 0)
    def _():
        acc_ref[...] = part

    @pl.when(i > 0)
    def _():
        acc_ref[...] += part

    @pl.when(i == _NSTEPS - 1)
    def _():
        pooled = acc_ref[...] * (1.0 / _SEQ)

        def _ln(h, g, b, eps=1e-5):
            m = jnp.mean(h, axis=-1, keepdims=True)
            v = jnp.mean((h - m) ** 2, axis=-1, keepdims=True)
            return (h - m) / jnp.sqrt(v + eps) * g + b

        h = jax.lax.dot_general(pooled, w1_ref[...], (((1,), (1,)), ((), ())),
                                preferred_element_type=jnp.float32) + b1_ref[...]
        h = jax.nn.relu(_ln(h, g1_ref[...], be1_ref[...]))
        h = jax.lax.dot_general(h, w2_ref[...], (((1,), (1,)), ((), ())),
                                preferred_element_type=jnp.float32) + b2_ref[...]
        h = jax.nn.relu(_ln(h, g2_ref[...], be2_ref[...]))
        scores = jax.lax.dot_general(h, w3_ref[...], (((1,), (1,)), ((), ())),
                                     preferred_element_type=jnp.float32) + b3_ref[...]

        scaled = scores - jnp.max(scores, axis=-1, keepdims=True)
        e = jnp.exp(scaled - jnp.max(scaled, axis=-1, keepdims=True))
        probs = e / jnp.sum(e, axis=-1, keepdims=True)

        pa = probs[:, :, None]
        pb = probs[:, None, :]
        ii = jax.lax.broadcasted_iota(jnp.int32, (1, _NUM_LAYERS, _NUM_LAYERS), 1)
        jj = jax.lax.broadcasted_iota(jnp.int32, (1, _NUM_LAYERS, _NUM_LAYERS), 2)
        beats = (pb > pa) | ((pb == pa) & (jj < ii))
        nbeat = jnp.sum(beats.astype(jnp.int32), axis=-1)
        o_ref[...] = (nbeat < 8).astype(jnp.float32)


def kernel(x, W1, b1, g1, be1, W2, b2, g2, be2, W3, b3):
    const = lambda s: pl.BlockSpec(s, lambda i: tuple(0 for _ in s))
    return pl.pallas_call(
        _fused_kernel,
        grid=(_NSTEPS,),
        in_specs=[pl.BlockSpec((_BATCH, _SBLK, _D), lambda i: (0, i, 0)),
                  const(W1.shape), const(b1.shape), const(g1.shape),
                  const(be1.shape), const(W2.shape), const(b2.shape),
                  const(g2.shape), const(be2.shape), const(W3.shape),
                  const(b3.shape)],
        out_specs=const((_BATCH, _NUM_LAYERS)),
        out_shape=jax.ShapeDtypeStruct((_BATCH, _NUM_LAYERS), jnp.float32),
        scratch_shapes=[pltpu.VMEM((_BATCH, _D), jnp.float32)],
    )(x, W1, b1, g1, be1, W2, b2, g2, be2, W3, b3)
